# baseline (speedup 1.0000x reference)
import jax
import jax.numpy as jnp
from jax.experimental import pallas as pl

N = 10000
E = 160000
C = 256
B = 64
L = 3
NC = 10


def _mlp_body(r_ref, w1_ref, b1_ref, w2_ref, b2_ref, w3_ref, b3_ref, o_ref):
    z = jnp.maximum(r_ref[...] @ w1_ref[...] + b1_ref[...], 0.0)
    z = jnp.maximum(z @ w2_ref[...] + b2_ref[...], 0.0)
    z = z @ w3_ref[...] + b3_ref[...]
    z = z - jnp.max(z, axis=-1, keepdims=True)
    z = z - jnp.log(jnp.sum(jnp.exp(z), axis=-1, keepdims=True))
    o_ref[...] = z


def _fa_mask_conv(h, x0, src, dst, edge_norm, mask, al, ar, eps):
    sl = h @ al
    sr = h @ ar
    alpha = jnp.tanh(sl[dst] + sr[src])
    coef = alpha * mask * edge_norm
    agg = jax.ops.segment_sum(coef[:, None] * h[src], dst, num_segments=N)
    return eps * x0 + agg


def _readout(x_out, batch):
    gm = jax.ops.segment_max(x_out, batch, num_segments=B)
    gm = jnp.where(jnp.isneginf(gm), 0.0, gm)
    cnt = jax.ops.segment_sum(jnp.ones((N,), dtype=jnp.float32), batch, num_segments=B)
    ga = jax.ops.segment_sum(x_out, batch, num_segments=B) / jnp.maximum(cnt, 1.0)[:, None]
    return jnp.concatenate([gm, ga], axis=1)


def kernel(x, edge_index, batch, homophily_mask, heterophily_mask, last_epoch,
           Wpre, bpre, att_l_hom, att_r_hom, att_l_het, att_r_het,
           eps_hom, eps_het, W1, b1, W2, b2, W3, b3):
    src = edge_index[0]
    dst = edge_index[1]
    deg = jax.ops.segment_sum(jnp.ones((E,), dtype=jnp.float32), dst, num_segments=N)
    deg = jnp.maximum(deg, 1.0)
    dinv = 1.0 / jnp.sqrt(deg)
    edge_norm = dinv[src] * dinv[dst]

    h = x.astype(jnp.float32) @ Wpre + bpre

    x_hom = jax.nn.relu(_fa_mask_conv(h, h, src, dst, edge_norm, homophily_mask,
                                      att_l_hom[0], att_r_hom[0], eps_hom[0]))
    x_het = jax.nn.relu(_fa_mask_conv(h, h, src, dst, edge_norm, heterophily_mask,
                                      att_l_het[0], att_r_het[0], eps_het[0]))
    x_out = x_hom + x_het
    readout = _readout(x_out, batch)
    for i in range(1, L):
        x_hom = jax.nn.relu(_fa_mask_conv(x_hom, h, src, dst, edge_norm, homophily_mask,
                                          att_l_hom[i], att_r_hom[i], eps_hom[i]))
        x_het = jax.nn.relu(_fa_mask_conv(x_het, h, src, dst, edge_norm, heterophily_mask,
                                          att_l_het[i], att_r_het[i], eps_het[i]))
        x_out = x_hom + x_het
        readout = readout + _readout(x_out, batch)

    z = pl.pallas_call(
        _mlp_body,
        out_shape=jax.ShapeDtypeStruct((B, NC), jnp.float32),
    )(readout, W1, b1, W2, b2, W3, b3)
    return z


# speedup vs baseline: 3.1383x; 3.1383x over previous
import functools

import jax
import jax.numpy as jnp
from jax import lax
from jax.experimental import pallas as pl
from jax.experimental.pallas import tpu as pltpu
from jax.experimental.pallas import tpu_sc as plsc

N = 10000
E = 160000
C = 256
CH = 128
B = 64
L = 3
NC = 10

NCORES = 2
NSUB = 16
LANES = 16
NP = 10240
NPT = NP // NSUB
EPT = E // NSUB
G1 = 80
NCH1 = EPT // G1
G3 = 64
NCH3 = E // G3
CPW = NCH3 // 32
RZ = 64

_SC_MESH = dict(core_axis_name="c", subcore_axis_name="s")


def _fast_rsqrt(x):
    xi = lax.bitcast_convert_type(x, jnp.int32)
    yi = jnp.int32(0x5F3759DF) - lax.shift_right_logical(xi, 1)
    y = lax.bitcast_convert_type(yi, jnp.float32)
    for _ in range(3):
        y = y * (1.5 - 0.5 * x * y * y)
    return y


def _prep_body(src_hbm, dst_hbm, mh_hbm, mt_hbm, ones_hbm, zeros_hbm,
               whom_hbm, whet_hbm,
               acc_sh, idx_v, idx2_v, idx3_v, idx4_v, m1_v, m2_v, w1_v, w2_v,
               ones_v, dbuf_v, dsr_v, ddr_v, sem):
    cid = lax.axis_index("c")
    sid = lax.axis_index("s")

    pltpu.sync_copy(ones_hbm, ones_v)

    def zero_chunk(k, carry):
        r0 = pl.multiple_of(sid * NPT + k * RZ, 8)
        pltpu.sync_copy(zeros_hbm.at[pl.ds(k * 0, RZ)], acc_sh.at[pl.ds(r0, RZ)])
        return carry

    lax.fori_loop(0, NPT // RZ, zero_chunk, 0)
    plsc.subcore_barrier()

    ebase = sid * EPT

    def deg_chunk(k, carry):
        pltpu.sync_copy(dst_hbm.at[pl.ds(ebase + k * G1, G1)], idx_v)
        pltpu.sync_copy(ones_v, acc_sh.at[idx_v], add=True)
        return carry

    lax.fori_loop(0, NCH1, deg_chunk, 0)
    plsc.subcore_barrier()

    def dinv_chunk(k, carry):
        r0 = pl.multiple_of(sid * NPT + k * RZ, 8)
        pltpu.sync_copy(acc_sh.at[pl.ds(r0, RZ)], dbuf_v)
        for r in range(RZ):
            y = _fast_rsqrt(jnp.maximum(dbuf_v[r, pl.ds(0, LANES)], 1.0))
            for q in range(8):
                dbuf_v[r, pl.ds(q * LANES, LANES)] = y
        pltpu.sync_copy(dbuf_v, acc_sh.at[pl.ds(r0, RZ)])
        return carry

    lax.fori_loop(0, NPT // RZ, dinv_chunk, 0)
    plsc.subcore_barrier()

    wid = cid * NSUB + sid
    nch = jnp.where(wid == 31, NCH3 - 31 * CPW, CPW)

    def w_chunk(k, carry):
        e0 = pl.multiple_of((wid * CPW + k) * G3, 64)
        pltpu.sync_copy(src_hbm.at[pl.ds(e0, G3)], idx3_v)
        pltpu.sync_copy(dst_hbm.at[pl.ds(e0, G3)], idx4_v)
        pltpu.sync_copy(mh_hbm.at[pl.ds(pl.multiple_of(e0 // 8, 8), G3 // 8)], m1_v)
        pltpu.sync_copy(mt_hbm.at[pl.ds(pl.multiple_of(e0 // 8, 8), G3 // 8)], m2_v)
        pltpu.async_copy(acc_sh.at[idx3_v], dsr_v, sem).wait()
        pltpu.async_copy(acc_sh.at[idx4_v], ddr_v, sem).wait()
        for g in range(G3):
            blk, lo = g // 8, (g % 8) * LANES
            nrm = dsr_v[g, pl.ds(0, LANES)] * ddr_v[g, pl.ds(0, LANES)]
            w1_v[blk, pl.ds(lo, LANES)] = nrm * m1_v[blk, pl.ds(lo, LANES)]
            w2_v[blk, pl.ds(lo, LANES)] = nrm * m2_v[blk, pl.ds(lo, LANES)]
        pltpu.sync_copy(w1_v, whom_hbm.at[pl.ds(pl.multiple_of(e0 // 8, 8), G3 // 8)])
        pltpu.sync_copy(w2_v, whet_hbm.at[pl.ds(pl.multiple_of(e0 // 8, 8), G3 // 8)])
        return carry

    lax.fori_loop(0, nch, w_chunk, 0)


def _sc_prep(src, dst, mh_rep, mt_rep):
    mesh = plsc.VectorSubcoreMesh(**_SC_MESH)
    ones = jnp.ones((G1, CH), jnp.float32)
    zeros = jnp.zeros((RZ, CH), jnp.float32)
    f = pl.kernel(
        _prep_body,
        out_type=(jax.ShapeDtypeStruct((E // 8, CH), jnp.float32),
                  jax.ShapeDtypeStruct((E // 8, CH), jnp.float32)),
        mesh=mesh,
        scratch_types=[
            pltpu.VMEM_SHARED((NP, CH), jnp.float32),
            pltpu.VMEM((G1,), jnp.int32),
            pltpu.VMEM((G1,), jnp.int32),
            pltpu.VMEM((G3,), jnp.int32),
            pltpu.VMEM((G3,), jnp.int32),
            pltpu.VMEM((G3 // 8, CH), jnp.float32),
            pltpu.VMEM((G3 // 8, CH), jnp.float32),
            pltpu.VMEM((G3 // 8, CH), jnp.float32),
            pltpu.VMEM((G3 // 8, CH), jnp.float32),
            pltpu.VMEM((G1, CH), jnp.float32),
            pltpu.VMEM((RZ, CH), jnp.float32),
            pltpu.VMEM((G3, CH), jnp.float32),
            pltpu.VMEM((G3, CH), jnp.float32),
            pltpu.SemaphoreType.DMA,
        ],
    )
    return f(src, dst, mh_rep, mt_rep, ones, zeros)


G2 = 64
CPC = NCH3 // NSUB


def _conv_body(src_hbm, dst_hbm, w_hbm, sl_hbm, sr_hbm, x0_hbm, xp_hbm,
               eps_hbm, xn_hbm,
               acc_sh, sidx_v, didx_v, m1_v, xbuf_v, slb_v, srb_v, eps_v, sem):
    cid = lax.axis_index("c")
    sid = lax.axis_index("s")
    coff = cid * NP

    pltpu.sync_copy(eps_hbm, eps_v)

    def init_chunk(k, carry):
        r0 = pl.multiple_of(sid * NPT + k * RZ, 8)
        pltpu.sync_copy(x0_hbm.at[pl.ds(coff + r0, RZ)], acc_sh.at[pl.ds(r0, RZ)])
        return carry

    lax.fori_loop(0, NPT // RZ, init_chunk, 0)
    plsc.subcore_barrier()

    nch = jnp.where(sid == NSUB - 1, NCH3 - (NSUB - 1) * CPC, CPC)

    def edge_chunk(k, carry):
        e0 = pl.multiple_of((sid * CPC + k) * G2, 64)
        b0 = pl.multiple_of(e0 // 8, 8)
        pltpu.sync_copy(src_hbm.at[pl.ds(e0, G2)], sidx_v)
        pltpu.sync_copy(dst_hbm.at[pl.ds(e0, G2)], didx_v)
        pltpu.sync_copy(w_hbm.at[pl.ds(b0, G2 // 8)], m1_v)
        pltpu.async_copy(sr_hbm.at[sidx_v], srb_v, sem).wait()
        pltpu.async_copy(sl_hbm.at[didx_v], slb_v, sem).wait()
        for t in range(G2 // LANES):
            sidx_v[pl.ds(t * LANES, LANES)] = (
                sidx_v[pl.ds(t * LANES, LANES)] + coff)
        pltpu.async_copy(xp_hbm.at[sidx_v], xbuf_v, sem).wait()
        for g in range(G2):
            blk, lo = g // 8, (g % 8) * LANES
            a = slb_v[g, pl.ds(0, LANES)] + srb_v[g, pl.ds(0, LANES)]
            t = jnp.exp(-2.0 * jnp.abs(a))
            th = (1.0 - t) / (1.0 + t)
            th = jnp.where(a < 0.0, -th, th)
            cvec = th * m1_v[blk, pl.ds(lo, LANES)]
            for q in range(8):
                xbuf_v[g, pl.ds(q * LANES, LANES)] = (
                    xbuf_v[g, pl.ds(q * LANES, LANES)] * cvec)
        pltpu.sync_copy(xbuf_v, acc_sh.at[didx_v], add=True)
        return carry

    lax.fori_loop(0, nch, edge_chunk, 0)
    plsc.subcore_barrier()

    em1 = eps_v[pl.ds(0, LANES)] - 1.0

    def ep_chunk(k, carry):
        r0 = pl.multiple_of(sid * NPT + k * RZ, 8)
        pltpu.sync_copy(acc_sh.at[pl.ds(r0, RZ)], slb_v)
        pltpu.sync_copy(x0_hbm.at[pl.ds(coff + r0, RZ)], xbuf_v)
        for r in range(RZ):
            for q in range(8):
                v = (slb_v[r, pl.ds(q * LANES, LANES)]
                     + em1 * xbuf_v[r, pl.ds(q * LANES, LANES)])
                slb_v[r, pl.ds(q * LANES, LANES)] = jnp.maximum(v, 0.0)
        pltpu.sync_copy(slb_v, xn_hbm.at[pl.ds(coff + r0, RZ)])
        return carry

    lax.fori_loop(0, NPT // RZ, ep_chunk, 0)


def _sc_conv(src, dst, w_rep, sl_tab, sr_tab, x0_st, xp_st, eps_rep):
    mesh = plsc.VectorSubcoreMesh(**_SC_MESH)
    f = pl.kernel(
        _conv_body,
        out_type=jax.ShapeDtypeStruct((2 * NP, CH), jnp.float32),
        mesh=mesh,
        scratch_types=[
            pltpu.VMEM_SHARED((NP, CH), jnp.float32),
            pltpu.VMEM((G2,), jnp.int32),
            pltpu.VMEM((G2,), jnp.int32),
            pltpu.VMEM((G2 // 8, CH), jnp.float32),
            pltpu.VMEM((G2, CH), jnp.float32),
            pltpu.VMEM((G2, CH), jnp.float32),
            pltpu.VMEM((G2, CH), jnp.float32),
            pltpu.VMEM((CH,), jnp.float32),
            pltpu.SemaphoreType.DMA,
        ],
    )
    return f(src, dst, w_rep, sl_tab, sr_tab, x0_st, xp_st, eps_rep)


def _tc_pre_body(x_ref, wp_ref, bp_ref, alh_ref, arh_ref, alt_ref, art_ref,
                 hl_ref, hh_ref, slh_ref, srh_ref, slt_ref, srt_ref):
    i = pl.program_id(0)
    h = jnp.dot(x_ref[...], wp_ref[...],
                preferred_element_type=jnp.float32) + bp_ref[...]
    rows = i * 640 + jax.lax.broadcasted_iota(jnp.int32, (640, 1), 0)
    h = jnp.where(rows < N, h, 0.0)
    hl_ref[...] = h[:, :CH]
    hh_ref[...] = h[:, CH:]
    for s_ref, a_ref in ((slh_ref, alh_ref), (srh_ref, arh_ref),
                         (slt_ref, alt_ref), (srt_ref, art_ref)):
        sv = jnp.dot(h, a_ref[0], preferred_element_type=jnp.float32)
        s_ref[...] = jnp.broadcast_to(sv[:, None], (640, CH))


def _tc_pre(x_pad, Wpre, bpre, al_h, ar_h, al_t, ar_t):
    out = jax.ShapeDtypeStruct((NP, CH), jnp.float32)
    blk = pl.BlockSpec((640, CH), lambda i: (i, 0))
    whole = lambda shp: pl.BlockSpec(shp, lambda i: tuple(0 for _ in shp))
    return pl.pallas_call(
        _tc_pre_body,
        grid=(NP // 640,),
        in_specs=[pl.BlockSpec((640, C), lambda i: (i, 0)),
                  whole((C, C)), whole((1, C)), whole((1, C)), whole((1, C)),
                  whole((1, C)), whole((1, C))],
        out_specs=[blk] * 6,
        out_shape=[out] * 6,
    )(x_pad, Wpre, bpre, al_h, ar_h, al_t, ar_t)


def _tc_dots_body(xhl_ref, xhh_ref, xtl_ref, xth_ref,
                  alh_ref, arh_ref, alt_ref, art_ref,
                  slh_ref, srh_ref, slt_ref, srt_ref):
    xh = jnp.concatenate([xhl_ref[...], xhh_ref[...]], axis=1)
    xt = jnp.concatenate([xtl_ref[...], xth_ref[...]], axis=1)
    for s_ref, a_ref, xf in ((slh_ref, alh_ref, xh), (srh_ref, arh_ref, xh),
                             (slt_ref, alt_ref, xt), (srt_ref, art_ref, xt)):
        sv = jnp.dot(xf, a_ref[0], preferred_element_type=jnp.float32)
        s_ref[...] = jnp.broadcast_to(sv[:, None], (640, CH))


def _tc_dots(xh_st, xt_st, al_h, ar_h, al_t, ar_t):
    out = jax.ShapeDtypeStruct((NP, CH), jnp.float32)
    blk = pl.BlockSpec((640, CH), lambda i: (i, 0))
    hi = pl.BlockSpec((640, CH), lambda i: (i + NP // 640, 0))
    whole = lambda shp: pl.BlockSpec(shp, lambda i: tuple(0 for _ in shp))
    return pl.pallas_call(
        _tc_dots_body,
        grid=(NP // 640,),
        in_specs=[blk, hi, blk, hi,
                  whole((1, C)), whole((1, C)), whole((1, C)), whole((1, C))],
        out_specs=[blk] * 4,
        out_shape=[out] * 4,
    )(xh_st, xh_st, xt_st, xt_st, al_h, ar_h, al_t, ar_t)


def _mlp_body(r_ref, w1_ref, b1_ref, w2_ref, b2_ref, w3_ref, b3_ref, o_ref):
    z = jnp.maximum(r_ref[...] @ w1_ref[...] + b1_ref[...], 0.0)
    z = jnp.maximum(z @ w2_ref[...] + b2_ref[...], 0.0)
    z = z @ w3_ref[...] + b3_ref[...]
    z = z - jnp.max(z, axis=-1, keepdims=True)
    z = z - jnp.log(jnp.sum(jnp.exp(z), axis=-1, keepdims=True))
    o_ref[...] = z


def _readout(x_out, batch):
    gm = jax.ops.segment_max(x_out, batch, num_segments=B)
    gm = jnp.where(jnp.isneginf(gm), 0.0, gm)
    cnt = jax.ops.segment_sum(jnp.ones((N,), dtype=jnp.float32), batch, num_segments=B)
    ga = jax.ops.segment_sum(x_out, batch, num_segments=B) / jnp.maximum(cnt, 1.0)[:, None]
    return jnp.concatenate([gm, ga], axis=1)


def kernel(x, edge_index, batch, homophily_mask, heterophily_mask, last_epoch,
           Wpre, bpre, att_l_hom, att_r_hom, att_l_het, att_r_het,
           eps_hom, eps_het, W1, b1, W2, b2, W3, b3):
    src = edge_index[0]
    dst = edge_index[1]

    mh_rep = jnp.broadcast_to(homophily_mask[:, None], (E, LANES)).reshape(E // 8, CH)
    mt_rep = jnp.broadcast_to(heterophily_mask[:, None], (E, LANES)).reshape(E // 8, CH)
    whom_rep, whet_rep = _sc_prep(src, dst, mh_rep, mt_rep)

    x_pad = jnp.pad(x.astype(jnp.float32), ((0, NP - N), (0, 0)))
    hl, hh, slh, srh, slt, srt = _tc_pre(
        x_pad, Wpre, bpre.reshape(1, C),
        att_l_hom[0].reshape(1, C), att_r_hom[0].reshape(1, C),
        att_l_het[0].reshape(1, C), att_r_het[0].reshape(1, C))
    h_st = jnp.concatenate([hl, hh], axis=0)

    xh_st, xt_st = h_st, h_st
    readout = None
    for i in range(L):
        eh = jnp.full((CH,), eps_hom[i], jnp.float32)
        et = jnp.full((CH,), eps_het[i], jnp.float32)
        xh_st = _sc_conv(src, dst, whom_rep, slh, srh, h_st, xh_st, eh)
        xt_st = _sc_conv(src, dst, whet_rep, slt, srt, h_st, xt_st, et)
        x_out = (jnp.concatenate([xh_st[:N], xh_st[NP:NP + N]], axis=1)
                 + jnp.concatenate([xt_st[:N], xt_st[NP:NP + N]], axis=1))
        r = _readout(x_out, batch)
        readout = r if readout is None else readout + r
        if i + 1 < L:
            slh, srh, slt, srt = _tc_dots(
                xh_st, xt_st,
                att_l_hom[i + 1].reshape(1, C), att_r_hom[i + 1].reshape(1, C),
                att_l_het[i + 1].reshape(1, C), att_r_het[i + 1].reshape(1, C))

    z = pl.pallas_call(
        _mlp_body,
        out_shape=jax.ShapeDtypeStruct((B, NC), jnp.float32),
    )(readout, W1, b1, W2, b2, W3, b3)
    return z


# speedup vs baseline: 3.8728x; 1.2340x over previous
import functools

import jax
import jax.numpy as jnp
from jax import lax
from jax.experimental import pallas as pl
from jax.experimental.pallas import tpu as pltpu
from jax.experimental.pallas import tpu_sc as plsc

N = 10000
E = 160000
C = 256
CH = 128
B = 64
L = 3
NC = 10

NCORES = 2
NSUB = 16
LANES = 16
NP = 10240
NPT = NP // NSUB
EPT = E // NSUB
G1 = 80
NCH1 = EPT // G1
G3 = 64
NCH3 = E // G3
CPW = NCH3 // 32
RZ = 64

_SC_MESH = dict(core_axis_name="c", subcore_axis_name="s")


def _fast_rsqrt(x):
    xi = lax.bitcast_convert_type(x, jnp.int32)
    yi = jnp.int32(0x5F3759DF) - lax.shift_right_logical(xi, 1)
    y = lax.bitcast_convert_type(yi, jnp.float32)
    for _ in range(3):
        y = y * (1.5 - 0.5 * x * y * y)
    return y


def _prep_body(src_hbm, dst_hbm, mh_hbm, mt_hbm, ones_hbm, zeros_hbm,
               whom_hbm, whet_hbm,
               acc_sh, idx_v, idx2_v, idx3_v, idx4_v, m1_v, m2_v, w1_v, w2_v,
               ones_v, dbuf_v, dsr_v, ddr_v, sem):
    cid = lax.axis_index("c")
    sid = lax.axis_index("s")

    pltpu.sync_copy(ones_hbm, ones_v)

    def zero_chunk(k, carry):
        r0 = pl.multiple_of(sid * NPT + k * RZ, 8)
        pltpu.sync_copy(zeros_hbm.at[pl.ds(k * 0, RZ)], acc_sh.at[pl.ds(r0, RZ)])
        return carry

    lax.fori_loop(0, NPT // RZ, zero_chunk, 0)
    plsc.subcore_barrier()

    ebase = sid * EPT

    def deg_chunk(k, carry):
        pltpu.sync_copy(dst_hbm.at[pl.ds(ebase + k * G1, G1)], idx_v)
        pltpu.sync_copy(ones_v, acc_sh.at[idx_v], add=True)
        return carry

    lax.fori_loop(0, NCH1, deg_chunk, 0)
    plsc.subcore_barrier()

    def dinv_chunk(k, carry):
        r0 = pl.multiple_of(sid * NPT + k * RZ, 8)
        pltpu.sync_copy(acc_sh.at[pl.ds(r0, RZ)], dbuf_v)
        for r in range(RZ):
            y = _fast_rsqrt(jnp.maximum(dbuf_v[r, pl.ds(0, LANES)], 1.0))
            for q in range(8):
                dbuf_v[r, pl.ds(q * LANES, LANES)] = y
        pltpu.sync_copy(dbuf_v, acc_sh.at[pl.ds(r0, RZ)])
        return carry

    lax.fori_loop(0, NPT // RZ, dinv_chunk, 0)
    plsc.subcore_barrier()

    wid = cid * NSUB + sid
    nch = jnp.where(wid == 31, NCH3 - 31 * CPW, CPW)

    def w_chunk(k, carry):
        e0 = pl.multiple_of((wid * CPW + k) * G3, 64)
        pltpu.sync_copy(src_hbm.at[pl.ds(e0, G3)], idx3_v)
        pltpu.sync_copy(dst_hbm.at[pl.ds(e0, G3)], idx4_v)
        pltpu.sync_copy(mh_hbm.at[pl.ds(pl.multiple_of(e0 // 8, 8), G3 // 8)], m1_v)
        pltpu.sync_copy(mt_hbm.at[pl.ds(pl.multiple_of(e0 // 8, 8), G3 // 8)], m2_v)
        pltpu.async_copy(acc_sh.at[idx3_v], dsr_v, sem).wait()
        pltpu.async_copy(acc_sh.at[idx4_v], ddr_v, sem).wait()
        for g in range(G3):
            blk, lo = g // 8, (g % 8) * LANES
            nrm = dsr_v[g, pl.ds(0, LANES)] * ddr_v[g, pl.ds(0, LANES)]
            w1_v[blk, pl.ds(lo, LANES)] = nrm * m1_v[blk, pl.ds(lo, LANES)]
            w2_v[blk, pl.ds(lo, LANES)] = nrm * m2_v[blk, pl.ds(lo, LANES)]
        pltpu.sync_copy(w1_v, whom_hbm.at[pl.ds(pl.multiple_of(e0 // 8, 8), G3 // 8)])
        pltpu.sync_copy(w2_v, whet_hbm.at[pl.ds(pl.multiple_of(e0 // 8, 8), G3 // 8)])
        return carry

    lax.fori_loop(0, nch, w_chunk, 0)


def _sc_prep(src, dst, mh_rep, mt_rep):
    mesh = plsc.VectorSubcoreMesh(**_SC_MESH)
    ones = jnp.ones((G1, CH), jnp.float32)
    zeros = jnp.zeros((RZ, CH), jnp.float32)
    f = pl.kernel(
        _prep_body,
        out_type=(jax.ShapeDtypeStruct((E // 8, CH), jnp.float32),
                  jax.ShapeDtypeStruct((E // 8, CH), jnp.float32)),
        mesh=mesh,
        scratch_types=[
            pltpu.VMEM_SHARED((NP, CH), jnp.float32),
            pltpu.VMEM((G1,), jnp.int32),
            pltpu.VMEM((G1,), jnp.int32),
            pltpu.VMEM((G3,), jnp.int32),
            pltpu.VMEM((G3,), jnp.int32),
            pltpu.VMEM((G3 // 8, CH), jnp.float32),
            pltpu.VMEM((G3 // 8, CH), jnp.float32),
            pltpu.VMEM((G3 // 8, CH), jnp.float32),
            pltpu.VMEM((G3 // 8, CH), jnp.float32),
            pltpu.VMEM((G1, CH), jnp.float32),
            pltpu.VMEM((RZ, CH), jnp.float32),
            pltpu.VMEM((G3, CH), jnp.float32),
            pltpu.VMEM((G3, CH), jnp.float32),
            pltpu.SemaphoreType.DMA,
        ],
    )
    return f(src, dst, mh_rep, mt_rep, ones, zeros)


G2 = 64
CPC = NCH3 // NSUB


def _conv_body(src_hbm, dst_hbm, w_hbm, sl_hbm, sr_hbm, x0_hbm, xp_hbm,
               eps_hbm, xn_hbm,
               acc_sh, sidx_v, sidx2_v, didx_v, m1_v, xbuf_v, slb_v, srb_v,
               eps_v, sem, sem2, sem3):
    cid = lax.axis_index("c")
    sid = lax.axis_index("s")
    coff = cid * NP

    pltpu.sync_copy(eps_hbm, eps_v)

    def init_chunk(k, carry):
        r0 = pl.multiple_of(sid * NPT + k * RZ, 8)
        pltpu.sync_copy(x0_hbm.at[pl.ds(coff + r0, RZ)], acc_sh.at[pl.ds(r0, RZ)])
        return carry

    lax.fori_loop(0, NPT // RZ, init_chunk, 0)
    plsc.subcore_barrier()

    nch = jnp.where(sid == NSUB - 1, NCH3 - (NSUB - 1) * CPC, CPC)

    def edge_chunk(k, carry):
        e0 = pl.multiple_of((sid * CPC + k) * G2, 64)
        b0 = pl.multiple_of(e0 // 8, 8)
        pltpu.sync_copy(src_hbm.at[pl.ds(e0, G2)], sidx_v)
        pltpu.sync_copy(dst_hbm.at[pl.ds(e0, G2)], didx_v)
        pltpu.sync_copy(w_hbm.at[pl.ds(b0, G2 // 8)], m1_v)
        cp1 = pltpu.async_copy(sr_hbm.at[sidx_v], srb_v, sem)
        cp2 = pltpu.async_copy(sl_hbm.at[didx_v], slb_v, sem2)
        for t in range(G2 // LANES):
            sidx2_v[pl.ds(t * LANES, LANES)] = (
                sidx_v[pl.ds(t * LANES, LANES)] + coff)
        cp3 = pltpu.async_copy(xp_hbm.at[sidx2_v], xbuf_v, sem3)
        cp1.wait()
        cp2.wait()
        cp3.wait()
        for g in range(G2):
            blk, lo = g // 8, (g % 8) * LANES
            a = slb_v[g, pl.ds(0, LANES)] + srb_v[g, pl.ds(0, LANES)]
            t = jnp.exp(-2.0 * jnp.abs(a))
            th = (1.0 - t) / (1.0 + t)
            th = jnp.where(a < 0.0, -th, th)
            cvec = th * m1_v[blk, pl.ds(lo, LANES)]
            for q in range(8):
                xbuf_v[g, pl.ds(q * LANES, LANES)] = (
                    xbuf_v[g, pl.ds(q * LANES, LANES)] * cvec)
        pltpu.sync_copy(xbuf_v, acc_sh.at[didx_v], add=True)
        return carry

    lax.fori_loop(0, nch, edge_chunk, 0)
    plsc.subcore_barrier()

    em1 = eps_v[pl.ds(0, LANES)] - 1.0

    def ep_chunk(k, carry):
        r0 = pl.multiple_of(sid * NPT + k * RZ, 8)
        pltpu.sync_copy(acc_sh.at[pl.ds(r0, RZ)], slb_v)
        pltpu.sync_copy(x0_hbm.at[pl.ds(coff + r0, RZ)], xbuf_v)
        for r in range(RZ):
            for q in range(8):
                v = (slb_v[r, pl.ds(q * LANES, LANES)]
                     + em1 * xbuf_v[r, pl.ds(q * LANES, LANES)])
                slb_v[r, pl.ds(q * LANES, LANES)] = jnp.maximum(v, 0.0)
        pltpu.sync_copy(slb_v, xn_hbm.at[pl.ds(coff + r0, RZ)])
        return carry

    lax.fori_loop(0, NPT // RZ, ep_chunk, 0)


def _sc_conv(src, dst, w_rep, sl_tab, sr_tab, x0_st, xp_st, eps_rep):
    mesh = plsc.VectorSubcoreMesh(**_SC_MESH)
    f = pl.kernel(
        _conv_body,
        out_type=jax.ShapeDtypeStruct((2 * NP, CH), jnp.float32),
        mesh=mesh,
        scratch_types=[
            pltpu.VMEM_SHARED((NP, CH), jnp.float32),
            pltpu.VMEM((G2,), jnp.int32),
            pltpu.VMEM((G2,), jnp.int32),
            pltpu.VMEM((G2,), jnp.int32),
            pltpu.VMEM((G2 // 8, CH), jnp.float32),
            pltpu.VMEM((G2, CH), jnp.float32),
            pltpu.VMEM((G2, CH), jnp.float32),
            pltpu.VMEM((G2, CH), jnp.float32),
            pltpu.VMEM((CH,), jnp.float32),
            pltpu.SemaphoreType.DMA,
            pltpu.SemaphoreType.DMA,
            pltpu.SemaphoreType.DMA,
        ],
    )
    return f(src, dst, w_rep, sl_tab, sr_tab, x0_st, xp_st, eps_rep)


def _tc_pre_body(x_ref, wp_ref, bp_ref, alh_ref, arh_ref, alt_ref, art_ref,
                 hl_ref, hh_ref, slh_ref, srh_ref, slt_ref, srt_ref):
    i = pl.program_id(0)
    h = jnp.dot(x_ref[...], wp_ref[...],
                preferred_element_type=jnp.float32) + bp_ref[...]
    rows = i * 640 + jax.lax.broadcasted_iota(jnp.int32, (640, 1), 0)
    h = jnp.where(rows < N, h, 0.0)
    hl_ref[...] = h[:, :CH]
    hh_ref[...] = h[:, CH:]
    for s_ref, a_ref in ((slh_ref, alh_ref), (srh_ref, arh_ref),
                         (slt_ref, alt_ref), (srt_ref, art_ref)):
        sv = jnp.dot(h, a_ref[0], preferred_element_type=jnp.float32)
        s_ref[...] = jnp.broadcast_to(sv[:, None], (640, CH))


def _tc_pre(x_pad, Wpre, bpre, al_h, ar_h, al_t, ar_t):
    out = jax.ShapeDtypeStruct((NP, CH), jnp.float32)
    blk = pl.BlockSpec((640, CH), lambda i: (i, 0))
    whole = lambda shp: pl.BlockSpec(shp, lambda i: tuple(0 for _ in shp))
    return pl.pallas_call(
        _tc_pre_body,
        grid=(NP // 640,),
        in_specs=[pl.BlockSpec((640, C), lambda i: (i, 0)),
                  whole((C, C)), whole((1, C)), whole((1, C)), whole((1, C)),
                  whole((1, C)), whole((1, C))],
        out_specs=[blk] * 6,
        out_shape=[out] * 6,
    )(x_pad, Wpre, bpre, al_h, ar_h, al_t, ar_t)


def _tc_dots_body(xhl_ref, xhh_ref, xtl_ref, xth_ref,
                  alh_ref, arh_ref, alt_ref, art_ref,
                  slh_ref, srh_ref, slt_ref, srt_ref):
    xh = jnp.concatenate([xhl_ref[...], xhh_ref[...]], axis=1)
    xt = jnp.concatenate([xtl_ref[...], xth_ref[...]], axis=1)
    for s_ref, a_ref, xf in ((slh_ref, alh_ref, xh), (srh_ref, arh_ref, xh),
                             (slt_ref, alt_ref, xt), (srt_ref, art_ref, xt)):
        sv = jnp.dot(xf, a_ref[0], preferred_element_type=jnp.float32)
        s_ref[...] = jnp.broadcast_to(sv[:, None], (640, CH))


def _tc_dots(xh_st, xt_st, al_h, ar_h, al_t, ar_t):
    out = jax.ShapeDtypeStruct((NP, CH), jnp.float32)
    blk = pl.BlockSpec((640, CH), lambda i: (i, 0))
    hi = pl.BlockSpec((640, CH), lambda i: (i + NP // 640, 0))
    whole = lambda shp: pl.BlockSpec(shp, lambda i: tuple(0 for _ in shp))
    return pl.pallas_call(
        _tc_dots_body,
        grid=(NP // 640,),
        in_specs=[blk, hi, blk, hi,
                  whole((1, C)), whole((1, C)), whole((1, C)), whole((1, C))],
        out_specs=[blk] * 4,
        out_shape=[out] * 4,
    )(xh_st, xh_st, xt_st, xt_st, al_h, ar_h, al_t, ar_t)


def _mlp_body(r_ref, w1_ref, b1_ref, w2_ref, b2_ref, w3_ref, b3_ref, o_ref):
    z = jnp.maximum(r_ref[...] @ w1_ref[...] + b1_ref[...], 0.0)
    z = jnp.maximum(z @ w2_ref[...] + b2_ref[...], 0.0)
    z = z @ w3_ref[...] + b3_ref[...]
    z = z - jnp.max(z, axis=-1, keepdims=True)
    z = z - jnp.log(jnp.sum(jnp.exp(z), axis=-1, keepdims=True))
    o_ref[...] = z


def _readout(x_out, batch):
    gm = jax.ops.segment_max(x_out, batch, num_segments=B,
                             indices_are_sorted=True)
    gm = jnp.where(jnp.isneginf(gm), 0.0, gm)
    cnt = jax.ops.segment_sum(jnp.ones((N,), dtype=jnp.float32), batch,
                              num_segments=B, indices_are_sorted=True)
    ga = (jax.ops.segment_sum(x_out, batch, num_segments=B,
                              indices_are_sorted=True)
          / jnp.maximum(cnt, 1.0)[:, None])
    return jnp.concatenate([gm, ga], axis=1)


def kernel(x, edge_index, batch, homophily_mask, heterophily_mask, last_epoch,
           Wpre, bpre, att_l_hom, att_r_hom, att_l_het, att_r_het,
           eps_hom, eps_het, W1, b1, W2, b2, W3, b3):
    src = edge_index[0]
    dst = edge_index[1]

    mh_rep = jnp.broadcast_to(homophily_mask[:, None], (E, LANES)).reshape(E // 8, CH)
    mt_rep = jnp.broadcast_to(heterophily_mask[:, None], (E, LANES)).reshape(E // 8, CH)
    whom_rep, whet_rep = _sc_prep(src, dst, mh_rep, mt_rep)

    x_pad = jnp.pad(x.astype(jnp.float32), ((0, NP - N), (0, 0)))
    hl, hh, slh, srh, slt, srt = _tc_pre(
        x_pad, Wpre, bpre.reshape(1, C),
        att_l_hom[0].reshape(1, C), att_r_hom[0].reshape(1, C),
        att_l_het[0].reshape(1, C), att_r_het[0].reshape(1, C))
    h_st = jnp.concatenate([hl, hh], axis=0)

    xh_st, xt_st = h_st, h_st
    readout = None
    for i in range(L):
        eh = jnp.full((CH,), eps_hom[i], jnp.float32)
        et = jnp.full((CH,), eps_het[i], jnp.float32)
        xh_st = _sc_conv(src, dst, whom_rep, slh, srh, h_st, xh_st, eh)
        xt_st = _sc_conv(src, dst, whet_rep, slt, srt, h_st, xt_st, et)
        x_out = (jnp.concatenate([xh_st[:N], xh_st[NP:NP + N]], axis=1)
                 + jnp.concatenate([xt_st[:N], xt_st[NP:NP + N]], axis=1))
        r = _readout(x_out, batch)
        readout = r if readout is None else readout + r
        if i + 1 < L:
            slh, srh, slt, srt = _tc_dots(
                xh_st, xt_st,
                att_l_hom[i + 1].reshape(1, C), att_r_hom[i + 1].reshape(1, C),
                att_l_het[i + 1].reshape(1, C), att_r_het[i + 1].reshape(1, C))

    z = pl.pallas_call(
        _mlp_body,
        out_shape=jax.ShapeDtypeStruct((B, NC), jnp.float32),
    )(readout, W1, b1, W2, b2, W3, b3)
    return z
